# baseline (device time: 21511 ns/iter reference)
import functools

import jax
import jax.numpy as jnp
from jax import lax
from jax.experimental import pallas as pl
from jax.experimental.pallas import tpu as pltpu

N_DEV = 16
N_TOK = 256
D_IN = 128
D_OUT = 256
N_EXP = 32
ROWS_PER_DEV = N_TOK // N_DEV
N_STEPS = 4


def kernel(x, router_W, route_idx, expert_W, shared_W):
    def body(x_ref, router_ref, ridx_ref, ew_ref, sw_ref, out_ref,
             acc_ref, recv_ref, send_sems, recv_sems):
        r = lax.axis_index("i")

        barrier_sem = pltpu.get_barrier_semaphore()
        for k in range(N_STEPS):
            pl.semaphore_signal(
                barrier_sem, inc=1,
                device_id=(jnp.bitwise_xor(r, 8 >> k),),
                device_id_type=pl.DeviceIdType.MESH,
            )
        pl.semaphore_wait(barrier_sem, N_STEPS)

        xb = x_ref[...].astype(jnp.bfloat16)

        scores = jnp.dot(xb, router_ref[...].astype(jnp.bfloat16),
                         preferred_element_type=jnp.float32)
        s_max = jnp.max(scores, axis=-1, keepdims=True)
        e_sc = jnp.exp(scores - s_max)
        probs = e_sc / jnp.sum(e_sc, axis=-1, keepdims=True)
        e_idx = ridx_ref[...]
        expert_iota = lax.broadcasted_iota(jnp.int32, (N_TOK, N_EXP), 1)
        gate = jnp.sum(jnp.where(expert_iota == e_idx, probs, 0.0),
                       axis=-1, keepdims=True)

        y0 = jnp.dot(xb, ew_ref[0].astype(jnp.bfloat16),
                     preferred_element_type=jnp.float32)
        y1 = jnp.dot(xb, ew_ref[1].astype(jnp.bfloat16),
                     preferred_element_type=jnp.float32)
        m0 = (e_idx == 2 * r).astype(jnp.float32)
        m1 = (e_idx == 2 * r + 1).astype(jnp.float32)
        acc_ref[...] = gate * (m0 * y0 + m1 * y1)

        for k in range(N_STEPS):
            b = 8 >> k
            half = 128 >> k
            start = ((r >> (N_STEPS - k)) << (N_STEPS - k)) * ROWS_PER_DEV
            keep_low = (r & b) == 0
            keep_start = start + jnp.where(keep_low, 0, half)
            send_start = start + jnp.where(keep_low, half, 0)
            rdma = pltpu.make_async_remote_copy(
                src_ref=acc_ref.at[pl.ds(send_start, half), :],
                dst_ref=recv_ref.at[k, pl.ds(0, half), :],
                send_sem=send_sems.at[k],
                recv_sem=recv_sems.at[k],
                device_id=(jnp.bitwise_xor(r, b),),
                device_id_type=pl.DeviceIdType.MESH,
            )
            rdma.start()
            rdma.wait()
            acc_ref[pl.ds(keep_start, half), :] += recv_ref[k, pl.ds(0, half), :]

        xs = x_ref[pl.ds(r * ROWS_PER_DEV, ROWS_PER_DEV), :].astype(jnp.bfloat16)
        shared = jnp.dot(xs, sw_ref[...].astype(jnp.bfloat16),
                         preferred_element_type=jnp.float32)
        out_ref[...] = acc_ref[pl.ds(r * ROWS_PER_DEV, ROWS_PER_DEV), :] + shared

        @functools.partial(pl.run_scoped,
                           second_barrier=pltpu.SemaphoreType.REGULAR)
        def _(second_barrier):
            for k in range(N_STEPS):
                pl.semaphore_signal(
                    second_barrier, inc=1,
                    device_id=(jnp.bitwise_xor(r, 8 >> k),),
                    device_id_type=pl.DeviceIdType.MESH,
                )
            pl.semaphore_wait(second_barrier, N_STEPS)

    return pl.pallas_call(
        body,
        out_shape=jax.ShapeDtypeStruct((ROWS_PER_DEV, D_OUT), jnp.float32),
        in_specs=[pl.BlockSpec(memory_space=pltpu.VMEM)] * 5,
        out_specs=pl.BlockSpec(memory_space=pltpu.VMEM),
        scratch_shapes=[
            pltpu.VMEM((N_TOK, D_OUT), jnp.float32),
            pltpu.VMEM((N_STEPS, 128, D_OUT), jnp.float32),
            pltpu.SemaphoreType.DMA((N_STEPS,)),
            pltpu.SemaphoreType.DMA((N_STEPS,)),
        ],
        compiler_params=pltpu.CompilerParams(collective_id=0),
    )(x, router_W, route_idx, expert_W, shared_W)


# device time: 17904 ns/iter; 1.2015x vs baseline; 1.2015x over previous
import functools

import jax
import jax.numpy as jnp
from jax import lax
from jax.experimental import pallas as pl
from jax.experimental.pallas import tpu as pltpu

N_DEV = 16
N_TOK = 256
D_IN = 128
D_OUT = 256
N_EXP = 32
ROWS_PER_DEV = N_TOK // N_DEV


def kernel(x, router_W, route_idx, expert_W, shared_W):
    def body(x_ref, router_ref, ridx_ref, ew_ref, sw_ref, out_ref,
             send_ref, recv_ref, send_sems, recv_sems):
        r = lax.axis_index("i")

        barrier_sem = pltpu.get_barrier_semaphore()
        for o in range(N_DEV):
            @pl.when(o != r)
            def _(o=o):
                pl.semaphore_signal(
                    barrier_sem, inc=1, device_id=(o,),
                    device_id_type=pl.DeviceIdType.MESH,
                )
        pl.semaphore_wait(barrier_sem, N_DEV - 1)

        xb = x_ref[...].astype(jnp.bfloat16)

        scores = jnp.dot(xb, router_ref[...].astype(jnp.bfloat16),
                         preferred_element_type=jnp.float32)
        s_max = jnp.max(scores, axis=-1, keepdims=True)
        e_sc = jnp.exp(scores - s_max)
        probs = e_sc / jnp.sum(e_sc, axis=-1, keepdims=True)
        e_idx = ridx_ref[...]
        expert_iota = lax.broadcasted_iota(jnp.int32, (N_TOK, N_EXP), 1)
        gate = jnp.sum(jnp.where(expert_iota == e_idx, probs, 0.0),
                       axis=-1, keepdims=True)

        y0 = jnp.dot(xb, ew_ref[0].astype(jnp.bfloat16),
                     preferred_element_type=jnp.float32)
        y1 = jnp.dot(xb, ew_ref[1].astype(jnp.bfloat16),
                     preferred_element_type=jnp.float32)
        m0 = (e_idx == 2 * r).astype(jnp.float32)
        m1 = (e_idx == 2 * r + 1).astype(jnp.float32)
        partial = gate * (m0 * y0 + m1 * y1)
        send_ref[...] = partial.astype(jnp.bfloat16).reshape(
            N_DEV, ROWS_PER_DEV, D_OUT)

        sends = []
        for j in range(N_DEV):
            rdma = pltpu.make_async_remote_copy(
                src_ref=send_ref.at[j],
                dst_ref=recv_ref.at[r],
                send_sem=send_sems.at[j],
                recv_sem=recv_sems.at[r],
                device_id=(j,),
                device_id_type=pl.DeviceIdType.MESH,
            )
            sends.append(rdma)

            @pl.when(j != r)
            def _(rdma=rdma):
                rdma.start()

        recv_ref[r] = send_ref[r]

        for s in range(N_DEV):
            recv = pltpu.make_async_remote_copy(
                src_ref=send_ref.at[s],
                dst_ref=recv_ref.at[s],
                send_sem=send_sems.at[s],
                recv_sem=recv_sems.at[s],
                device_id=(s,),
                device_id_type=pl.DeviceIdType.MESH,
            )

            @pl.when(s != r)
            def _(recv=recv):
                recv.wait_recv()

        moe = jnp.sum(recv_ref[...].astype(jnp.float32), axis=0)

        xs = x_ref[pl.ds(r * ROWS_PER_DEV, ROWS_PER_DEV), :].astype(jnp.bfloat16)
        shared = jnp.dot(xs, sw_ref[...].astype(jnp.bfloat16),
                         preferred_element_type=jnp.float32)
        out_ref[...] = moe + shared

        for j in range(N_DEV):
            @pl.when(j != r)
            def _(rdma=sends[j]):
                rdma.wait_send()

        @functools.partial(pl.run_scoped,
                           second_barrier=pltpu.SemaphoreType.REGULAR)
        def _(second_barrier):
            for o in range(N_DEV):
                @pl.when(o != r)
                def _(o=o):
                    pl.semaphore_signal(
                        second_barrier, inc=1, device_id=(o,),
                        device_id_type=pl.DeviceIdType.MESH,
                    )
            pl.semaphore_wait(second_barrier, N_DEV - 1)

    return pl.pallas_call(
        body,
        out_shape=jax.ShapeDtypeStruct((ROWS_PER_DEV, D_OUT), jnp.float32),
        in_specs=[pl.BlockSpec(memory_space=pltpu.VMEM)] * 5,
        out_specs=pl.BlockSpec(memory_space=pltpu.VMEM),
        scratch_shapes=[
            pltpu.VMEM((N_DEV, ROWS_PER_DEV, D_OUT), jnp.bfloat16),
            pltpu.VMEM((N_DEV, ROWS_PER_DEV, D_OUT), jnp.bfloat16),
            pltpu.SemaphoreType.DMA((N_DEV,)),
            pltpu.SemaphoreType.DMA((N_DEV,)),
        ],
        compiler_params=pltpu.CompilerParams(collective_id=0),
    )(x, router_W, route_idx, expert_W, shared_W)


# device time: 12282 ns/iter; 1.7514x vs baseline; 1.4577x over previous
import jax
import jax.numpy as jnp
from jax import lax
from jax.experimental import pallas as pl
from jax.experimental.pallas import tpu as pltpu

N_DEV = 16
N_TOK = 256
D_IN = 128
D_OUT = 256
N_EXP = 32
ROWS_PER_DEV = N_TOK // N_DEV


def kernel(x, router_W, route_idx, expert_W, shared_W):
    def body(x_ref, router_ref, ridx_ref, ew_ref, sw_ref, out_ref,
             send_ref, recv_ref, send_sems, recv_sems):
        r = lax.axis_index("i")

        barrier_sem = pltpu.get_barrier_semaphore()
        for o in range(N_DEV):
            @pl.when(o != r)
            def _(o=o):
                pl.semaphore_signal(
                    barrier_sem, inc=1, device_id=(o,),
                    device_id_type=pl.DeviceIdType.MESH,
                )

        xb = x_ref[...].astype(jnp.bfloat16)

        scores = jnp.dot(xb, router_ref[...].astype(jnp.bfloat16),
                         preferred_element_type=jnp.float32)
        s_max = jnp.max(scores, axis=-1, keepdims=True)
        e_sc = jnp.exp(scores - s_max)
        probs = e_sc / jnp.sum(e_sc, axis=-1, keepdims=True)
        e_idx = ridx_ref[...]
        expert_iota = lax.broadcasted_iota(jnp.int32, (N_TOK, N_EXP), 1)
        gate = jnp.sum(jnp.where(expert_iota == e_idx, probs, 0.0),
                       axis=-1, keepdims=True)

        y0 = jnp.dot(xb, ew_ref[0].astype(jnp.bfloat16),
                     preferred_element_type=jnp.float32)
        y1 = jnp.dot(xb, ew_ref[1].astype(jnp.bfloat16),
                     preferred_element_type=jnp.float32)
        m0 = (e_idx == 2 * r).astype(jnp.float32)
        m1 = (e_idx == 2 * r + 1).astype(jnp.float32)
        partial = gate * (m0 * y0 + m1 * y1)
        send_ref[...] = partial.astype(jnp.bfloat16).reshape(
            N_DEV, ROWS_PER_DEV, D_OUT)

        pl.semaphore_wait(barrier_sem, N_DEV - 1)

        sends = []
        for j in range(N_DEV):
            rdma = pltpu.make_async_remote_copy(
                src_ref=send_ref.at[j],
                dst_ref=recv_ref.at[r],
                send_sem=send_sems.at[j],
                recv_sem=recv_sems.at[r],
                device_id=(j,),
                device_id_type=pl.DeviceIdType.MESH,
            )
            sends.append(rdma)

            @pl.when(j != r)
            def _(rdma=rdma):
                rdma.start()

        recv_ref[r] = send_ref[r]
        xs = x_ref[pl.ds(r * ROWS_PER_DEV, ROWS_PER_DEV), :].astype(jnp.bfloat16)
        shared = jnp.dot(xs, sw_ref[...].astype(jnp.bfloat16),
                         preferred_element_type=jnp.float32)

        for s in range(N_DEV):
            recv = pltpu.make_async_remote_copy(
                src_ref=send_ref.at[s],
                dst_ref=recv_ref.at[s],
                send_sem=send_sems.at[s],
                recv_sem=recv_sems.at[s],
                device_id=(s,),
                device_id_type=pl.DeviceIdType.MESH,
            )

            @pl.when(s != r)
            def _(recv=recv):
                recv.wait_recv()

        moe = jnp.sum(recv_ref[...].astype(jnp.float32), axis=0)
        out_ref[...] = moe + shared

        for j in range(N_DEV):
            @pl.when(j != r)
            def _(rdma=sends[j]):
                rdma.wait_send()

    return pl.pallas_call(
        body,
        out_shape=jax.ShapeDtypeStruct((ROWS_PER_DEV, D_OUT), jnp.float32),
        in_specs=[pl.BlockSpec(memory_space=pltpu.VMEM)] * 5,
        out_specs=pl.BlockSpec(memory_space=pltpu.VMEM),
        scratch_shapes=[
            pltpu.VMEM((N_DEV, ROWS_PER_DEV, D_OUT), jnp.bfloat16),
            pltpu.VMEM((N_DEV, ROWS_PER_DEV, D_OUT), jnp.bfloat16),
            pltpu.SemaphoreType.DMA((N_DEV,)),
            pltpu.SemaphoreType.DMA((N_DEV,)),
        ],
        compiler_params=pltpu.CompilerParams(collective_id=0),
    )(x, router_W, route_idx, expert_W, shared_W)
